# baseline (speedup 1.0000x reference)
import functools

import numpy as np

import jax
import jax.numpy as jnp
from jax import lax
from jax.experimental import pallas as pl
from jax.experimental.pallas import tpu as pltpu
from jax.experimental.pallas import tpu_sc as plsc

N = 10000
IN = 128
HID = 256
OUT = 128
D = 128
N_PAD = 10240
NC = 2
NS = 16
L = 16
NW = NC * NS
CHUNK = 128
ROWS_PER_TILE = N_PAD // NS
BN = 2560


_Z = np.int32(0)


def _i32(v):
    return jnp.asarray(v, jnp.int32)


def _sc_body(nchunks, compute_deg, *refs):
    if compute_deg:
        (table, idxc, zrows, zflat,
         agg_out, deg_out, acc, ibuf, rows, cnt, sem_g, sem_i, sem_s) = refs
    else:
        (table, idxc, zrows,
         agg_out, acc, ibuf, rows, sem_g, sem_i, sem_s) = refs

    c = _i32(lax.axis_index("c"))
    s = _i32(lax.axis_index("s"))
    wid = _i32(s * NC + c)
    base = _i32(s * ROWS_PER_TILE)

    pltpu.sync_copy(zrows, acc.at[pl.ds(base, ROWS_PER_TILE)])
    if compute_deg:
        pltpu.sync_copy(zflat, cnt)
    plsc.subcore_barrier()

    ones = jnp.ones((L,), jnp.float32)
    z0, z1 = _i32(0), _i32(1)
    NQ = 1
    QR = CHUNK // NQ

    def fire_gather(b):
        for q in range(NQ):
            pltpu.async_copy(
                table.at[ibuf.at[_i32(b), z0, pl.ds(q * QR, QR)]],
                rows.at[_i32(b), pl.ds(q * QR, QR)], sem_g)

    def wait_gather(b):
        for q in range(NQ):
            pltpu.make_async_copy(
                table.at[ibuf.at[_i32(b), z0, pl.ds(q * QR, QR)]],
                rows.at[_i32(b), pl.ds(q * QR, QR)], sem_g).wait()

    pltpu.sync_copy(idxc.at[wid, z0], ibuf.at[z0])
    fire_gather(0)
    if nchunks > 1:
        pltpu.async_copy(idxc.at[wid, z1], ibuf.at[z1], sem_i)

    def do_chunk(j, b):
        j = _i32(j)
        nb = 1 - b
        @pl.when(j + 1 < nchunks)
        def _():
            pltpu.make_async_copy(idxc.at[wid, j + 1], ibuf.at[_i32(nb)],
                                  sem_i).wait()
        wait_gather(b)
        @pl.when(j >= 1)
        def _():
            pltpu.make_async_copy(rows.at[_i32(nb)],
                                  acc.at[ibuf.at[_i32(nb), z1]], sem_s).wait()
        @pl.when(j + 1 < nchunks)
        def _():
            fire_gather(nb)
        if compute_deg:
            for t in range(CHUNK // L):
                dv = ibuf[b, 1, pl.ds(t * L, L)]
                plsc.addupdate_scatter(cnt, [dv], ones)
        pltpu.async_copy(rows.at[_i32(b)], acc.at[ibuf.at[_i32(b), z1]],
                         sem_s, add=True)
        @pl.when(j + 2 < nchunks)
        def _():
            pltpu.async_copy(idxc.at[wid, j + 2], ibuf.at[_i32(b)], sem_i)

    def pair_body(jj, _):
        do_chunk(jj * 2, 0)
        do_chunk(jj * 2 + 1, 1)
        return _

    lax.fori_loop(jnp.int32(0), jnp.int32(nchunks // 2), pair_body,
                  jnp.int32(0))
    if nchunks % 2:
        do_chunk(nchunks - 1, 0)

    lb = _i32((nchunks - 1) % 2)
    pltpu.make_async_copy(rows.at[lb], acc.at[ibuf.at[lb, z1]], sem_s).wait()

    plsc.subcore_barrier()
    pltpu.sync_copy(acc.at[pl.ds(base, ROWS_PER_TILE)],
                    agg_out.at[c, pl.ds(base, ROWS_PER_TILE)])
    if compute_deg:
        pltpu.sync_copy(cnt, deg_out.at[wid])


def _make_sc_segsum(nchunks, compute_deg):
    scratch = [
        pltpu.VMEM_SHARED((N_PAD, D), jnp.float32),
        pltpu.VMEM((2, 2, CHUNK), jnp.int32),
        pltpu.VMEM((2, CHUNK, D), jnp.float32),
    ]
    out_type = [jax.ShapeDtypeStruct((NC, N_PAD, D), jnp.float32)]
    if compute_deg:
        scratch.append(pltpu.VMEM((N_PAD,), jnp.float32))
        out_type.append(jax.ShapeDtypeStruct((NW, N_PAD), jnp.float32))
    scratch.append(pltpu.SemaphoreType.DMA)
    scratch.append(pltpu.SemaphoreType.DMA)
    scratch.append(pltpu.SemaphoreType.DMA)
    mesh = plsc.VectorSubcoreMesh(core_axis_name="c", subcore_axis_name="s")
    return pl.kernel(
        functools.partial(_sc_body, nchunks, compute_deg),
        out_type=out_type, mesh=mesh, scratch_types=scratch,
        compiler_params=pltpu.CompilerParams(needs_layout_passes=False),
        name=f"sc_segsum_deg{int(compute_deg)}",
    )


_DN = (((1,), (1,)), ((), ()))


def _dot(a, b):
    ah = a.astype(jnp.bfloat16)
    al = (a - ah.astype(jnp.float32)).astype(jnp.bfloat16)
    bh = b.astype(jnp.bfloat16)
    bl = (b - bh.astype(jnp.float32)).astype(jnp.bfloat16)

    def d(u, v):
        return lax.dot_general(u, v, _DN, preferred_element_type=jnp.float32)

    return d(ah, bh) + d(al, bh) + d(ah, bl)


def _tc0_body(x, w1r, b1, xr_out):
    xr_out[...] = _dot(x[...], w1r[...]) + b1[...]


def _mean(aggp, degp):
    agg = aggp[0] + aggp[1]
    deg = jnp.sum(degp[...], axis=0)
    rdeg = 1.0 / jnp.maximum(deg, 1.0)
    return agg * rdeg[:, None], rdeg


def _tc1_body(aggp, degp, xr, w1l, w2l, h_out, p_out):
    mean, _ = _mean(aggp, degp)
    h = jnp.maximum(_dot(mean, w1l[...]) + xr[...], 0.0)
    h_out[...] = h
    p_out[...] = _dot(h, w2l[...])


def _tcr2_body(h, w2r, r2_out):
    r2_out[...] = _dot(h[...], w2r[...])


def _tc2_body(aggp, degp, r2, b2, out):
    mean, _ = _mean(aggp, degp)
    out[...] = mean + r2[...] + b2[...]


def _row_block_call(body, ins, outs, name):
    grid = N_PAD // BN

    def spec(shape):
        if shape[0] == NC:
            return pl.BlockSpec((NC, BN, shape[2]), lambda i: (_Z, i, _Z))
        if shape[0] == NW:
            return pl.BlockSpec((NW, BN), lambda i: (_Z, i))
        if shape[0] == N_PAD:
            return pl.BlockSpec((BN, shape[1]), lambda i: (i, _Z))
        return pl.BlockSpec(shape, lambda i: tuple(_Z for _ in shape))

    return pl.pallas_call(
        body,
        grid=(grid,),
        in_specs=[spec(a.shape) for a in ins],
        out_specs=[spec(s) for s in outs],
        out_shape=[jax.ShapeDtypeStruct(s, jnp.float32) for s in outs],
        name=name,
    )(*ins)


def kernel(x, edge_index, W1l, W1r, b1, W2l, W2r, b2):
    out_dtype = jnp.result_type(x.dtype, W1l.dtype, b1.dtype)
    W1l, W1r, b1, W2l, W2r, b2 = (
        a.astype(jnp.float32) for a in (W1l, W1r, b1, W2l, W2r, b2))
    x = x.astype(jnp.float32)
    E = edge_index.shape[1]
    nchunks = -(-E // (NW * CHUNK))
    e_pad = NW * nchunks * CHUNK
    src = edge_index[0].astype(jnp.int32)
    dst = edge_index[1].astype(jnp.int32)
    npad_e = e_pad - E
    pad_iota = jnp.arange(npad_e, dtype=jnp.int32)
    src = jnp.concatenate([src, pad_iota % N]).reshape(NW, nchunks, CHUNK)
    dst = jnp.concatenate(
        [dst, N + pad_iota % (N_PAD - N)]).reshape(NW, nchunks, CHUNK)
    idxc = jnp.stack([src, dst], axis=2)
    xp = jnp.zeros((N_PAD, IN), jnp.float32).at[:N].set(x)
    zrows = jnp.zeros((ROWS_PER_TILE, D), jnp.float32)
    zflat = jnp.zeros((N_PAD,), jnp.float32)

    sc_a = _make_sc_segsum(nchunks, True)
    sc_b = _make_sc_segsum(nchunks, False)

    agg1p, degp = sc_a(xp, idxc, zrows, zflat)
    (xr,) = _row_block_call(_tc0_body, (xp, W1r, b1.reshape(1, HID)),
                            [(N_PAD, HID)], "sage_tc0")
    h, p = _row_block_call(_tc1_body, (agg1p, degp, xr, W1l, W2l),
                           [(N_PAD, HID), (N_PAD, OUT)], "sage_tc1")
    (agg2p,) = sc_b(p, idxc, zrows)
    (r2,) = _row_block_call(_tcr2_body, (h, W2r), [(N_PAD, OUT)], "sage_tcr2")
    (out,) = _row_block_call(_tc2_body, (agg2p, degp, r2, b2.reshape(1, OUT)),
                             [(N_PAD, OUT)], "sage_tc2")
    return out[:N].astype(out_dtype)


# speedup vs baseline: 1.0183x; 1.0183x over previous
import functools

import numpy as np

import jax
import jax.numpy as jnp
from jax import lax
from jax.experimental import pallas as pl
from jax.experimental.pallas import tpu as pltpu
from jax.experimental.pallas import tpu_sc as plsc

N = 10000
IN = 128
HID = 256
OUT = 128
D = 128
N_PAD = 10240
NC = 2
NS = 16
L = 16
NW = NC * NS
CHUNK = 128
ROWS_PER_TILE = N_PAD // NS
BN = 1280


_Z = np.int32(0)


def _i32(v):
    return jnp.asarray(v, jnp.int32)


def _sc_body(nchunks, compute_deg, *refs):
    if compute_deg:
        (table, idxc, zrows, zflat,
         agg_out, deg_out, acc, ibuf, rows, cnt, sem_g, sem_i, sem_s) = refs
    else:
        (table, idxc, zrows,
         agg_out, acc, ibuf, rows, sem_g, sem_i, sem_s) = refs

    c = _i32(lax.axis_index("c"))
    s = _i32(lax.axis_index("s"))
    wid = _i32(s * NC + c)
    base = _i32(s * ROWS_PER_TILE)

    pltpu.sync_copy(zrows, acc.at[pl.ds(base, ROWS_PER_TILE)])
    if compute_deg:
        pltpu.sync_copy(zflat, cnt)
    plsc.subcore_barrier()

    ones = jnp.ones((L,), jnp.float32)
    z0, z1 = _i32(0), _i32(1)
    NQ = 1
    QR = CHUNK // NQ

    def fire_gather(b):
        for q in range(NQ):
            pltpu.async_copy(
                table.at[ibuf.at[_i32(b), z0, pl.ds(q * QR, QR)]],
                rows.at[_i32(b), pl.ds(q * QR, QR)], sem_g)

    def wait_gather(b):
        for q in range(NQ):
            pltpu.make_async_copy(
                table.at[ibuf.at[_i32(b), z0, pl.ds(q * QR, QR)]],
                rows.at[_i32(b), pl.ds(q * QR, QR)], sem_g).wait()

    row0 = _i32(wid * nchunks)

    def fetch_idx(j, b):
        pltpu.async_copy(idxc.at[z0, row0 + j], ibuf.at[_i32(b), z0], sem_i)
        pltpu.async_copy(idxc.at[z1, row0 + j], ibuf.at[_i32(b), z1], sem_i)

    def wait_idx(j, b):
        pltpu.make_async_copy(idxc.at[z0, row0 + j], ibuf.at[_i32(b), z0],
                              sem_i).wait()
        pltpu.make_async_copy(idxc.at[z1, row0 + j], ibuf.at[_i32(b), z1],
                              sem_i).wait()

    fetch_idx(z0, 0)
    wait_idx(z0, 0)
    fire_gather(0)
    if nchunks > 1:
        fetch_idx(z1, 1)

    def do_chunk(j, b):
        j = _i32(j)
        nb = 1 - b
        @pl.when(j + 1 < nchunks)
        def _():
            wait_idx(j + 1, nb)
        wait_gather(b)
        @pl.when(j >= 1)
        def _():
            pltpu.make_async_copy(rows.at[_i32(nb)],
                                  acc.at[ibuf.at[_i32(nb), z1]], sem_s).wait()
        @pl.when(j + 1 < nchunks)
        def _():
            fire_gather(nb)
        if compute_deg:
            for t in range(CHUNK // L):
                dv = ibuf[b, 1, pl.ds(t * L, L)]
                plsc.addupdate_scatter(cnt, [dv], ones)
        pltpu.async_copy(rows.at[_i32(b)], acc.at[ibuf.at[_i32(b), z1]],
                         sem_s, add=True)
        @pl.when(j + 2 < nchunks)
        def _():
            fetch_idx(j + 2, b)

    def pair_body(jj, _):
        do_chunk(jj * 2, 0)
        do_chunk(jj * 2 + 1, 1)
        return _

    lax.fori_loop(jnp.int32(0), jnp.int32(nchunks // 2), pair_body,
                  jnp.int32(0))
    if nchunks % 2:
        do_chunk(nchunks - 1, 0)

    lb = _i32((nchunks - 1) % 2)
    pltpu.make_async_copy(rows.at[lb], acc.at[ibuf.at[lb, z1]], sem_s).wait()

    plsc.subcore_barrier()
    pltpu.sync_copy(acc.at[pl.ds(base, ROWS_PER_TILE)],
                    agg_out.at[c, pl.ds(base, ROWS_PER_TILE)])
    if compute_deg:
        pltpu.sync_copy(cnt, deg_out.at[wid])


def _make_sc_segsum(nchunks, compute_deg):
    scratch = [
        pltpu.VMEM_SHARED((N_PAD, D), jnp.float32),
        pltpu.VMEM((2, 2, CHUNK), jnp.int32),
        pltpu.VMEM((2, CHUNK, D), jnp.float32),
    ]
    out_type = [jax.ShapeDtypeStruct((NC, N_PAD, D), jnp.float32)]
    if compute_deg:
        scratch.append(pltpu.VMEM((N_PAD,), jnp.float32))
        out_type.append(jax.ShapeDtypeStruct((NW, N_PAD), jnp.float32))
    scratch.append(pltpu.SemaphoreType.DMA)
    scratch.append(pltpu.SemaphoreType.DMA)
    scratch.append(pltpu.SemaphoreType.DMA)
    mesh = plsc.VectorSubcoreMesh(core_axis_name="c", subcore_axis_name="s")
    return pl.kernel(
        functools.partial(_sc_body, nchunks, compute_deg),
        out_type=out_type, mesh=mesh, scratch_types=scratch,
        compiler_params=pltpu.CompilerParams(needs_layout_passes=False),
        name=f"sc_segsum_deg{int(compute_deg)}",
    )


_DN = (((1,), (1,)), ((), ()))


def _dot(a, b):
    ah = a.astype(jnp.bfloat16)
    al = (a - ah.astype(jnp.float32)).astype(jnp.bfloat16)
    bh = b.astype(jnp.bfloat16)
    bl = (b - bh.astype(jnp.float32)).astype(jnp.bfloat16)

    def d(u, v):
        return lax.dot_general(u, v, _DN, preferred_element_type=jnp.float32)

    return d(ah, bh) + d(al, bh) + d(ah, bl)


def _tc0_body(x, w1r, b1, xr_out):
    xr_out[...] = _dot(x[...], w1r[...]) + b1[...]


def _mean(aggp, degp):
    agg = aggp[0] + aggp[1]
    deg = jnp.sum(degp[...], axis=0)
    rdeg = 1.0 / jnp.maximum(deg, 1.0)
    return agg * rdeg[:, None], rdeg


def _tc1_body(aggp, degp, xr, w1l, w2l, h_out, p_out):
    mean, _ = _mean(aggp, degp)
    h = jnp.maximum(_dot(mean, w1l[...]) + xr[...], 0.0)
    h_out[...] = h
    p_out[...] = _dot(h, w2l[...])


def _tcr2_body(h, w2r, r2_out):
    r2_out[...] = _dot(h[...], w2r[...])


def _tc2_body(aggp, degp, r2, b2, out):
    mean, _ = _mean(aggp, degp)
    out[...] = mean + r2[...] + b2[...]


def _row_block_call(body, ins, outs, name):
    grid = N_PAD // BN

    def spec(shape):
        if shape[0] == NC:
            return pl.BlockSpec((NC, BN, shape[2]), lambda i: (_Z, i, _Z))
        if shape[0] == NW:
            return pl.BlockSpec((NW, BN), lambda i: (_Z, i))
        if shape[0] == N_PAD:
            return pl.BlockSpec((BN, shape[1]), lambda i: (i, _Z))
        return pl.BlockSpec(shape, lambda i: tuple(_Z for _ in shape))

    return pl.pallas_call(
        body,
        grid=(grid,),
        in_specs=[spec(a.shape) for a in ins],
        out_specs=[spec(s) for s in outs],
        out_shape=[jax.ShapeDtypeStruct(s, jnp.float32) for s in outs],
        name=name,
    )(*ins)


def kernel(x, edge_index, W1l, W1r, b1, W2l, W2r, b2):
    out_dtype = jnp.result_type(x.dtype, W1l.dtype, b1.dtype)
    W1l, W1r, b1, W2l, W2r, b2 = (
        a.astype(jnp.float32) for a in (W1l, W1r, b1, W2l, W2r, b2))
    x = x.astype(jnp.float32)
    E = edge_index.shape[1]
    nchunks = -(-E // (NW * CHUNK))
    e_pad = NW * nchunks * CHUNK
    src = edge_index[0].astype(jnp.int32)
    dst = edge_index[1].astype(jnp.int32)
    npad_e = e_pad - E
    pad_iota = jnp.arange(npad_e, dtype=jnp.int32)
    padcols = jnp.stack([pad_iota % N, N + pad_iota % (N_PAD - N)])
    idxc = jnp.concatenate(
        [jnp.stack([src, dst]), padcols], axis=1,
    ).reshape(2, NW * nchunks, CHUNK)
    xp = jnp.zeros((N_PAD, IN), jnp.float32).at[:N].set(x)
    zrows = jnp.zeros((ROWS_PER_TILE, D), jnp.float32)
    zflat = jnp.zeros((N_PAD,), jnp.float32)

    sc_a = _make_sc_segsum(nchunks, True)
    sc_b = _make_sc_segsum(nchunks, False)

    agg1p, degp = sc_a(xp, idxc, zrows, zflat)
    (xr,) = _row_block_call(_tc0_body, (xp, W1r, b1.reshape(1, HID)),
                            [(N_PAD, HID)], "sage_tc0")
    h, p = _row_block_call(_tc1_body, (agg1p, degp, xr, W1l, W2l),
                           [(N_PAD, HID), (N_PAD, OUT)], "sage_tc1")
    (agg2p,) = sc_b(p, idxc, zrows)
    (r2,) = _row_block_call(_tcr2_body, (h, W2r), [(N_PAD, OUT)], "sage_tcr2")
    (out,) = _row_block_call(_tc2_body, (agg2p, degp, r2, b2.reshape(1, OUT)),
                             [(N_PAD, OUT)], "sage_tc2")
    return out[:N].astype(out_dtype)
